# baseline (device time: 11849 ns/iter reference)
import jax
import jax.numpy as jnp
from jax import lax
from jax.experimental import pallas as pl
from jax.experimental.pallas import tpu as pltpu

M = 512
N = 1024
HALF = N // 2

F = 160
FWD_CUTS = (0, 48, 96, 128, 160)
REST = M - 2 * F
DIRECT = F + REST
NF = len(FWD_CUTS) - 1


def kernel(x):
    def body(
        x_ref,
        out_ref,
        send_z,
        recv_z,
        recv_x,
        z_send_sems,
        z_recv_sems,
        x_send_sems,
        x_recv_sems,
    ):
        my_x = lax.axis_index("x")
        my_y = lax.axis_index("y")
        my_z = lax.axis_index("z")
        peer_z = (my_x, my_y, 1 - my_z)
        peer_x = (1 - my_x, my_y, my_z)

        barrier_sem = pltpu.get_barrier_semaphore()
        for nbr in (peer_z, peer_x):
            pl.semaphore_signal(
                barrier_sem,
                inc=1,
                device_id=nbr,
                device_id_type=pl.DeviceIdType.MESH,
            )

        other_off = (1 - my_z) * HALF
        fwd_row = my_x * F
        send_z[0:F] = x_ref[0, pl.ds(fwd_row, F), pl.ds(other_off, HALF)].astype(
            jnp.bfloat16
        )
        pl.semaphore_wait(barrier_sem, 2)

        def z_chunk(lo, hi, sem_idx):
            return pltpu.make_async_remote_copy(
                src_ref=send_z.at[lo:hi],
                dst_ref=recv_z.at[lo:hi],
                send_sem=z_send_sems.at[sem_idx],
                recv_sem=z_recv_sems.at[sem_idx],
                device_id=peer_z,
                device_id_type=pl.DeviceIdType.MESH,
            )

        z_fwd = []
        for i in range(NF):
            r = z_chunk(FWD_CUTS[i], FWD_CUTS[i + 1], i)
            r.start()
            z_fwd.append(r)

        send_z[F:DIRECT] = x_ref[
            0, pl.ds(2 * F, REST), pl.ds(other_off, HALF)
        ].astype(jnp.bfloat16)
        zc = z_chunk(F, DIRECT, NF)
        zc.start()

        my_off = my_z * HALF
        out_ref[...] = x_ref[0, :, pl.ds(my_off, HALF)]

        def x_fwd(lo, hi, sem_idx):
            return pltpu.make_async_remote_copy(
                src_ref=recv_z.at[lo:hi],
                dst_ref=recv_x.at[lo:hi],
                send_sem=x_send_sems.at[sem_idx],
                recv_sem=x_recv_sems.at[sem_idx],
                device_id=peer_x,
                device_id_type=pl.DeviceIdType.MESH,
            )

        fwds = []
        for i in range(NF):
            z_fwd[i].wait()
            f = x_fwd(FWD_CUTS[i], FWD_CUTS[i + 1], i)
            f.start()
            fwds.append(f)

        zc.wait()
        out_ref[pl.ds(fwd_row, F)] = out_ref[pl.ds(fwd_row, F)] + recv_z[
            0:F
        ].astype(jnp.float32)
        out_ref[2 * F : M] = out_ref[2 * F : M] + recv_z[F:DIRECT].astype(
            jnp.float32
        )

        for i in range(NF):
            fwds[i].wait_send()
        for i in range(NF):
            rx = pltpu.make_async_remote_copy(
                src_ref=recv_x.at[FWD_CUTS[i] : FWD_CUTS[i + 1]],
                dst_ref=recv_x.at[FWD_CUTS[i] : FWD_CUTS[i + 1]],
                send_sem=x_send_sems.at[i],
                recv_sem=x_recv_sems.at[i],
                device_id=peer_x,
                device_id_type=pl.DeviceIdType.MESH,
            )
            rx.wait_recv()
        nbr_row = (1 - my_x) * F
        out_ref[pl.ds(nbr_row, F)] = out_ref[pl.ds(nbr_row, F)] + recv_x[
            0:F
        ].astype(jnp.float32)

    return pl.pallas_call(
        body,
        out_shape=jax.ShapeDtypeStruct((M, HALF), jnp.float32),
        in_specs=[pl.BlockSpec(memory_space=pltpu.VMEM)],
        out_specs=pl.BlockSpec(memory_space=pltpu.VMEM),
        scratch_shapes=[
            pltpu.VMEM((DIRECT, HALF), jnp.bfloat16),
            pltpu.VMEM((DIRECT, HALF), jnp.bfloat16),
            pltpu.VMEM((F, HALF), jnp.bfloat16),
            pltpu.SemaphoreType.DMA((NF + 1,)),
            pltpu.SemaphoreType.DMA((NF + 1,)),
            pltpu.SemaphoreType.DMA((NF,)),
            pltpu.SemaphoreType.DMA((NF,)),
        ],
        compiler_params=pltpu.CompilerParams(collective_id=0),
    )(x)
